# baseline (speedup 1.0000x reference)
import functools

import jax
import jax.numpy as jnp
from jax import lax
from jax.experimental import pallas as pl
from jax.experimental.pallas import tpu as pltpu
from jax.experimental.pallas import tpu_sc as plsc

N_X = 16384
N_CB = 8192
D = 32
TILE_X = 2048
GRID = N_X // TILE_X


def _dist_argmin_body(scale_ref, x_ref, cb_ref, idx_ref, cbs_ref):
    s = scale_ref[0, 0]
    cbs = cb_ref[...] * s

    @pl.when(pl.program_id(0) == 0)
    def _():
        cbs_ref[...] = cbs

    x = x_ref[...]
    x2 = jnp.sum(x * x, axis=1, keepdims=True)
    c2 = jnp.sum(cbs * cbs, axis=1)[None, :]
    g2 = lax.dot_general(
        x * 2.0, cbs, (((1,), (1,)), ((), ())), preferred_element_type=jnp.float32
    )
    dist = x2 + c2 - g2
    idx_ref[...] = jnp.argmin(dist, axis=1).astype(jnp.int32)


_dist_argmin = pl.pallas_call(
    _dist_argmin_body,
    grid=(GRID,),
    in_specs=[
        pl.BlockSpec(memory_space=pltpu.SMEM),
        pl.BlockSpec((TILE_X, D), lambda i: (i, 0)),
        pl.BlockSpec((N_CB, D), lambda i: (0, 0)),
    ],
    out_specs=[
        pl.BlockSpec((TILE_X,), lambda i: (i,)),
        pl.BlockSpec((N_CB, D), lambda i: (0, 0)),
    ],
    out_shape=[
        jax.ShapeDtypeStruct((N_X,), jnp.int32),
        jax.ShapeDtypeStruct((N_CB, D), jnp.float32),
    ],
)

_NC = 2
_NS = 16
_NW = _NC * _NS
_BPW = N_X // _NW


@functools.cache
def _make_sc_gather():
    @functools.partial(
        pl.kernel,
        mesh=plsc.VectorSubcoreMesh(core_axis_name="c", subcore_axis_name="s"),
        out_type=jax.ShapeDtypeStruct((N_X, D), jnp.float32),
        scratch_types=[
            pltpu.VMEM((_BPW,), jnp.int32),
            pltpu.VMEM((_BPW, D), jnp.float32),
            pltpu.SemaphoreType.DMA,
        ],
        compiler_params=pltpu.CompilerParams(use_tc_tiling_on_sc=False),
    )
    def _sc_gather(cbs_hbm, idx_hbm, out_hbm, idx_v, rows_v, sem):
        wid = lax.axis_index("s") * _NC + lax.axis_index("c")
        base = wid * _BPW
        pltpu.sync_copy(idx_hbm.at[pl.ds(base, _BPW)], idx_v)
        pltpu.async_copy(cbs_hbm.at[idx_v], rows_v, sem).wait()
        pltpu.sync_copy(rows_v, out_hbm.at[pl.ds(base, _BPW)])

    return _sc_gather


def kernel(x, codebook, scale):
    idx, cbs = _dist_argmin(scale.reshape(1, 1), x, codebook)
    return _make_sc_gather()(cbs, idx)


# speedup vs baseline: 1.0049x; 1.0049x over previous
import functools

import jax
import jax.numpy as jnp
from jax import lax
from jax.experimental import pallas as pl
from jax.experimental.pallas import tpu as pltpu
from jax.experimental.pallas import tpu_sc as plsc

N_X = 16384
N_CB = 8192
D = 32
TILE_X = 1024
GRID = N_X // TILE_X


def _dist_argmin_body(scale_ref, x_ref, cb_ref, idx_ref, cbs_ref):
    s = scale_ref[0, 0]
    cbs = cb_ref[...] * s

    @pl.when(pl.program_id(0) == 0)
    def _():
        cbs_ref[...] = cbs

    x = x_ref[...]
    x2 = jnp.sum(x * x, axis=1, keepdims=True)
    c2 = jnp.sum(cbs * cbs, axis=1)[None, :]
    g2 = lax.dot_general(
        x * 2.0, cbs, (((1,), (1,)), ((), ())), preferred_element_type=jnp.float32
    )
    dist = x2 + c2 - g2
    idx_ref[...] = jnp.argmin(dist, axis=1).astype(jnp.int32)


_dist_argmin = pl.pallas_call(
    _dist_argmin_body,
    grid=(GRID,),
    in_specs=[
        pl.BlockSpec(memory_space=pltpu.SMEM),
        pl.BlockSpec((TILE_X, D), lambda i: (i, 0)),
        pl.BlockSpec((N_CB, D), lambda i: (0, 0)),
    ],
    out_specs=[
        pl.BlockSpec((TILE_X,), lambda i: (i,)),
        pl.BlockSpec((N_CB, D), lambda i: (0, 0)),
    ],
    out_shape=[
        jax.ShapeDtypeStruct((N_X,), jnp.int32),
        jax.ShapeDtypeStruct((N_CB, D), jnp.float32),
    ],
)

_NC = 2
_NS = 16
_NW = _NC * _NS
_BPW = N_X // _NW


@functools.cache
def _make_sc_gather():
    @functools.partial(
        pl.kernel,
        mesh=plsc.VectorSubcoreMesh(core_axis_name="c", subcore_axis_name="s"),
        out_type=jax.ShapeDtypeStruct((N_X, D), jnp.float32),
        scratch_types=[
            pltpu.VMEM((_BPW,), jnp.int32),
            pltpu.VMEM((_BPW, D), jnp.float32),
            pltpu.SemaphoreType.DMA,
        ],
        compiler_params=pltpu.CompilerParams(use_tc_tiling_on_sc=False),
    )
    def _sc_gather(cbs_hbm, idx_hbm, out_hbm, idx_v, rows_v, sem):
        wid = lax.axis_index("s") * _NC + lax.axis_index("c")
        base = wid * _BPW
        pltpu.sync_copy(idx_hbm.at[pl.ds(base, _BPW)], idx_v)
        pltpu.async_copy(cbs_hbm.at[idx_v], rows_v, sem).wait()
        pltpu.sync_copy(rows_v, out_hbm.at[pl.ds(base, _BPW)])

    return _sc_gather


def kernel(x, codebook, scale):
    idx, cbs = _dist_argmin(scale.reshape(1, 1), x, codebook)
    return _make_sc_gather()(cbs, idx)
